# Initial kernel scaffold; baseline (speedup 1.0000x reference)
import functools

import jax
import jax.numpy as jnp
from jax import lax
from jax.experimental import pallas as pl
from jax.experimental.pallas import tpu as pltpu
from jax.experimental.pallas import tpu_sc as plsc

B = 1024
S = 200
H = 128
TOK = B * S

NC = 2
NS = 16
NW = NC * NS
CPW = TOK // NW
CH = 128
NCH = CPW // CH

_MESH = plsc.VectorSubcoreMesh(core_axis_name="c", subcore_axis_name="s")


def _sc_gather(table, idx3):
    @functools.partial(
        pl.kernel,
        mesh=_MESH,
        out_type=jax.ShapeDtypeStruct((TOK, H), jnp.float32),
        scratch_types=[
            pltpu.VMEM((NCH, CH), jnp.int32),
            pltpu.VMEM((CH, H), jnp.float32),
            pltpu.SemaphoreType.DMA,
        ],
    )
    def k(table_hbm, idx_hbm, out_hbm, idx_v, rows_v, sem):
        wid = lax.axis_index("s") * NC + lax.axis_index("c")
        pltpu.sync_copy(idx_hbm.at[wid], idx_v)

        @pl.loop(0, NCH)
        def _(j):
            pltpu.async_copy(table_hbm.at[idx_v.at[j]], rows_v, sem).wait()
            pltpu.sync_copy(rows_v, out_hbm.at[pl.ds(wid * CPW + j * CH, CH)])

    return k(table, idx3)


def _ln_body(g_ref, tt_ref, pos_ref, type_ref, gam_ref, bet_ref, o_ref):
    x = g_ref[...]
    tt = tt_ref[...]
    pos = pos_ref[...][None, :, :]
    t0 = type_ref[0, :][None, None, :]
    t1 = type_ref[1, :][None, None, :]
    ty = jnp.where((tt == 1)[:, :, None], t1, t0)
    x = x + pos + ty
    mu = jnp.mean(x, axis=-1, keepdims=True)
    d = x - mu
    var = jnp.mean(d * d, axis=-1, keepdims=True)
    inv = lax.rsqrt(var + 1e-5)
    gam = gam_ref[0][None, None, :]
    bet = bet_ref[0][None, None, :]
    o_ref[...] = d * inv * gam + bet


_BB = 16


def _ln_call(g3, tt, pos, type_emb, gamma2, beta2):
    grid = (B // _BB,)
    return pl.pallas_call(
        _ln_body,
        grid=grid,
        in_specs=[
            pl.BlockSpec((_BB, S, H), lambda i: (i, 0, 0)),
            pl.BlockSpec((_BB, S), lambda i: (i, 0)),
            pl.BlockSpec((S, H), lambda i: (0, 0)),
            pl.BlockSpec((2, H), lambda i: (0, 0)),
            pl.BlockSpec((1, H), lambda i: (0, 0)),
            pl.BlockSpec((1, H), lambda i: (0, 0)),
        ],
        out_specs=pl.BlockSpec((_BB, S, H), lambda i: (i, 0, 0)),
        out_shape=jax.ShapeDtypeStruct((B, S, H), jnp.float32),
    )(g3, tt, pos, type_emb, gamma2, beta2)


def kernel(input_ids, token_type_ids, token_embedding, pos_embedding,
           type_embedding, ln_gamma, ln_beta):
    idx3 = input_ids.astype(jnp.int32).reshape(NW, NCH, CH)
    gathered = _sc_gather(token_embedding, idx3)
    g3 = gathered.reshape(B, S, H)
    out = _ln_call(
        g3,
        token_type_ids.astype(jnp.int32),
        pos_embedding[:S],
        type_embedding,
        ln_gamma.reshape(1, H),
        ln_beta.reshape(1, H),
    )
    return out


# speedup vs baseline: 8.3695x; 8.3695x over previous
import functools

import jax
import jax.numpy as jnp
from jax import lax
from jax.experimental import pallas as pl
from jax.experimental.pallas import tpu as pltpu
from jax.experimental.pallas import tpu_sc as plsc

B = 1024
S = 200
H = 128
TOK = B * S

NC = 2
NS = 16
NW = NC * NS
CPW = TOK // NW
CH = 128
NCH = CPW // CH

_MESH = plsc.VectorSubcoreMesh(core_axis_name="c", subcore_axis_name="s")


def _sc_gather(table, idx3):
    @functools.partial(
        pl.kernel,
        mesh=_MESH,
        out_type=jax.ShapeDtypeStruct((TOK, H), jnp.float32),
        scratch_types=[
            pltpu.VMEM((NCH, CH), jnp.int32),
            pltpu.VMEM((CH, H), jnp.float32),
            pltpu.SemaphoreType.DMA,
        ],
    )
    def k(table_hbm, idx_hbm, out_hbm, idx_v, rows_v, sem):
        wid = lax.axis_index("s") * NC + lax.axis_index("c")
        pltpu.sync_copy(idx_hbm.at[wid], idx_v)

        @pl.loop(0, NCH)
        def _(j):
            pltpu.async_copy(table_hbm.at[idx_v.at[j]], rows_v, sem).wait()
            pltpu.sync_copy(rows_v, out_hbm.at[pl.ds(wid * CPW + j * CH, CH)])

    return k(table, idx3)


def _ln_body(g_ref, tt_ref, pos_ref, t0_ref, t1_ref, gam_ref, bet_ref, o_ref):
    x = g_ref[...]
    ttf = tt_ref[...].astype(jnp.float32)
    pos = pos_ref[...]
    t0 = t0_ref[...]
    t1 = t1_ref[...]
    x = x + pos + t0 + ttf * (t1 - t0)
    mu = jnp.mean(x, axis=-1, keepdims=True)
    d = x - mu
    var = jnp.mean(d * d, axis=-1, keepdims=True)
    inv = lax.rsqrt(var + 1e-5)
    o_ref[...] = d * inv * gam_ref[...] + bet_ref[...]


_BB = 16


def _ln_call(g3, tt3, pos3, t0_3, t1_3, gam3, bet3):
    grid = (B // _BB,)
    return pl.pallas_call(
        _ln_body,
        grid=grid,
        in_specs=[
            pl.BlockSpec((_BB, S, H), lambda i: (i, 0, 0)),
            pl.BlockSpec((_BB, S, 1), lambda i: (i, 0, 0)),
            pl.BlockSpec((1, S, H), lambda i: (0, 0, 0)),
            pl.BlockSpec((1, 1, H), lambda i: (0, 0, 0)),
            pl.BlockSpec((1, 1, H), lambda i: (0, 0, 0)),
            pl.BlockSpec((1, 1, H), lambda i: (0, 0, 0)),
            pl.BlockSpec((1, 1, H), lambda i: (0, 0, 0)),
        ],
        out_specs=pl.BlockSpec((_BB, S, H), lambda i: (i, 0, 0)),
        out_shape=jax.ShapeDtypeStruct((B, S, H), jnp.float32),
    )(g3, tt3, pos3, t0_3, t1_3, gam3, bet3)


def kernel(input_ids, token_type_ids, token_embedding, pos_embedding,
           type_embedding, ln_gamma, ln_beta):
    idx3 = input_ids.astype(jnp.int32).reshape(NW, NCH, CH)
    gathered = _sc_gather(token_embedding, idx3)
    g3 = gathered.reshape(B, S, H)
    out = _ln_call(
        g3,
        token_type_ids.astype(jnp.int32).reshape(B, S, 1),
        pos_embedding[:S].reshape(1, S, H),
        type_embedding[0].reshape(1, 1, H),
        type_embedding[1].reshape(1, 1, H),
        ln_gamma.reshape(1, 1, H),
        ln_beta.reshape(1, 1, H),
    )
    return out


# speedup vs baseline: 8.7548x; 1.0460x over previous
import functools

import jax
import jax.numpy as jnp
from jax import lax
from jax.experimental import pallas as pl
from jax.experimental.pallas import tpu as pltpu
from jax.experimental.pallas import tpu_sc as plsc

B = 1024
S = 200
H = 128
TOK = B * S

NC = 2
NS = 16
NW = NC * NS
CPW = TOK // NW
CH = 128
NCH = CPW // CH

_MESH = plsc.VectorSubcoreMesh(core_axis_name="c", subcore_axis_name="s")


def _sc_gather(table, idx3):
    @functools.partial(
        pl.kernel,
        mesh=_MESH,
        out_type=jax.ShapeDtypeStruct((TOK, H), jnp.float32),
        scratch_types=[
            pltpu.VMEM((NCH, CH), jnp.int32),
            pltpu.VMEM((CH, H), jnp.float32),
            pltpu.VMEM((CH, H), jnp.float32),
            pltpu.SemaphoreType.DMA,
            pltpu.SemaphoreType.DMA,
        ],
    )
    def k(table_hbm, idx_hbm, out_hbm, idx_v, rows0, rows1, sg, sw):
        wid = lax.axis_index("s") * NC + lax.axis_index("c")
        base = wid * CPW
        pltpu.sync_copy(idx_hbm.at[wid], idx_v)
        rows = (rows0, rows1)

        def out_at(j):
            return out_hbm.at[pl.ds(base + j * CH, CH)]

        cg = [pltpu.async_copy(table_hbm.at[idx_v.at[b]], rows[b], sg)
              for b in range(2)]
        for b in range(2):
            cg[b].wait()
            pltpu.async_copy(rows[b], out_at(b), sw)

        @pl.loop(2, NCH, step=2)
        def _(j):
            c = []
            for b in range(2):
                pltpu.make_async_copy(rows[b], out_at(j - 2 + b), sw).wait()
                c.append(pltpu.async_copy(
                    table_hbm.at[idx_v.at[j + b]], rows[b], sg))
            for b in range(2):
                c[b].wait()
                pltpu.async_copy(rows[b], out_at(j + b), sw)

        for b in range(2):
            pltpu.make_async_copy(rows[b], out_at(NCH - 2 + b), sw).wait()

    return k(table, idx3)


def _ln_body(g_ref, tt_ref, pos_ref, t0_ref, t1_ref, gam_ref, bet_ref, o_ref):
    x = g_ref[...]
    ttf = tt_ref[...].astype(jnp.float32)
    pos = pos_ref[...]
    t0 = t0_ref[...]
    t1 = t1_ref[...]
    x = x + pos + t0 + ttf * (t1 - t0)
    mu = jnp.mean(x, axis=-1, keepdims=True)
    d = x - mu
    var = jnp.mean(d * d, axis=-1, keepdims=True)
    inv = lax.rsqrt(var + 1e-5)
    o_ref[...] = d * inv * gam_ref[...] + bet_ref[...]


_BB = 16


def _ln_call(g3, tt3, pos3, t0_3, t1_3, gam3, bet3):
    grid = (B // _BB,)
    return pl.pallas_call(
        _ln_body,
        grid=grid,
        in_specs=[
            pl.BlockSpec((_BB, S, H), lambda i: (i, 0, 0)),
            pl.BlockSpec((_BB, S, 1), lambda i: (i, 0, 0)),
            pl.BlockSpec((1, S, H), lambda i: (0, 0, 0)),
            pl.BlockSpec((1, 1, H), lambda i: (0, 0, 0)),
            pl.BlockSpec((1, 1, H), lambda i: (0, 0, 0)),
            pl.BlockSpec((1, 1, H), lambda i: (0, 0, 0)),
            pl.BlockSpec((1, 1, H), lambda i: (0, 0, 0)),
        ],
        out_specs=pl.BlockSpec((_BB, S, H), lambda i: (i, 0, 0)),
        out_shape=jax.ShapeDtypeStruct((B, S, H), jnp.float32),
    )(g3, tt3, pos3, t0_3, t1_3, gam3, bet3)


def kernel(input_ids, token_type_ids, token_embedding, pos_embedding,
           type_embedding, ln_gamma, ln_beta):
    idx3 = input_ids.astype(jnp.int32).reshape(NW, NCH, CH)
    gathered = _sc_gather(token_embedding, idx3)
    g3 = gathered.reshape(B, S, H)
    out = _ln_call(
        g3,
        token_type_ids.astype(jnp.int32).reshape(B, S, 1),
        pos_embedding[:S].reshape(1, S, H),
        type_embedding[0].reshape(1, 1, H),
        type_embedding[1].reshape(1, 1, H),
        ln_gamma.reshape(1, 1, H),
        ln_beta.reshape(1, 1, H),
    )
    return out
